# Initial kernel scaffold; baseline (speedup 1.0000x reference)
import jax
import jax.numpy as jnp
from jax.experimental import pallas as pl


def kernel(x, pos, W0, b0, W1, b1, W2, b2, W3, b3, g0, be0, g1, be1, g2, be2):
    raise NotImplementedError("write your pallas kernel here")


# speedup vs baseline: 4.1472x; 4.1472x over previous
import functools

import jax
import jax.numpy as jnp
from jax import lax
from jax.experimental import pallas as pl
from jax.experimental.pallas import tpu as pltpu
from jax.experimental.pallas import tpu_sc as plsc

_N = 10000
_NPAD = 10240
_RB = 400
_EPS = 1e-5

_NC, _NS, _L = 2, 16, 16
_NW = _NC * _NS
_BW = _NPAD // _NW
_CH = 64
_NCH = _BW // _CH


def _knn_body(pr2_ref, sqr_ref, pc2_ref, sqc_ref, idx_ref, cross_scr):
    cross_scr[...] = jnp.dot(pr2_ref[...], pc2_ref[...],
                             preferred_element_type=jnp.float32)
    lane = lax.broadcasted_iota(jnp.int32, (8, 128), 1)
    inf = jnp.float32(jnp.inf)

    def row_group(g, _):
        r0 = pl.multiple_of(g * 8, 8)
        sqr8 = sqr_ref[pl.ds(r0, 8), :]

        def chunk(s, carry):
            mv, iv = carry
            c0 = pl.multiple_of(s * 128, 128)
            cr = cross_scr[pl.ds(r0, 8), pl.ds(c0, 128)]
            sc = sqc_ref[:, pl.ds(c0, 128)]
            nv = (sqr8 + sc) - 2.0 * cr
            ni = lane + s * 128
            mv2, iv2 = [], []
            for l in range(6):
                c = nv < mv[l]
                mv2.append(jnp.where(c, nv, mv[l]))
                iv2.append(jnp.where(c, ni, iv[l]))
                nv, ni = (jnp.where(c, mv[l], nv),
                          jnp.where(c, iv[l], ni))
            return tuple(mv2), tuple(iv2)

        m0 = tuple(jnp.full((8, 128), inf, jnp.float32) for _ in range(6))
        i0 = tuple(jnp.full((8, 128), 2 ** 30, jnp.int32) for _ in range(6))
        mv, iv = lax.fori_loop(0, _NPAD // 128, chunk, (m0, i0))

        vals = jnp.concatenate(mv, axis=1)
        idxs = jnp.concatenate(iv, axis=1)
        for t in range(6):
            m = jnp.min(vals, axis=1, keepdims=True)
            am = jnp.min(jnp.where(vals == m, idxs, jnp.int32(2 ** 30)),
                         axis=1, keepdims=True)
            idx_ref[pl.ds(r0, 8), t:t + 1] = am
            if t < 5:
                vals = jnp.where((vals == m) & (idxs == am), inf, vals)
        idx_ref[pl.ds(r0, 8), 6:7] = jnp.zeros((8, 1), jnp.int32)
        idx_ref[pl.ds(r0, 8), 7:8] = jnp.zeros((8, 1), jnp.int32)
        return 0

    lax.fori_loop(0, _RB // 8, row_group, 0)


_knn_call = pl.pallas_call(
    _knn_body,
    grid=(_N // _RB,),
    in_specs=[
        pl.BlockSpec((_RB, 2), lambda i: (i, 0)),
        pl.BlockSpec((_RB, 1), lambda i: (i, 0)),
        pl.BlockSpec((2, _NPAD), lambda i: (0, 0)),
        pl.BlockSpec((1, _NPAD), lambda i: (0, 0)),
    ],
    out_specs=pl.BlockSpec((_RB, 8), lambda i: (i, 0)),
    out_shape=jax.ShapeDtypeStruct((_N, 8), jnp.int32),
    scratch_shapes=[pltpu.VMEM((_RB, _NPAD), jnp.float32)],
)


def _knn_idx(pos):
    p = pos[:, :2]
    sq = jnp.sum(p * p, axis=1)
    pc2 = jnp.concatenate(
        [p.T, jnp.zeros((2, _NPAD - _N), jnp.float32)], axis=1)
    sqc = jnp.concatenate(
        [sq[None, :], jnp.full((1, _NPAD - _N), jnp.inf, jnp.float32)], axis=1)
    return _knn_call(p, sq[:, None], pc2, sqc)


def _mm_body(x_ref, w_ref, o_ref):
    o_ref[...] = jnp.dot(x_ref[...], w_ref[...],
                         preferred_element_type=jnp.float32)


def _mm(x, w):
    return pl.pallas_call(
        _mm_body,
        out_shape=jax.ShapeDtypeStruct((x.shape[0], w.shape[1]), jnp.float32),
    )(x, w)


def _bnmm_body(u_ref, g_ref, be_ref, w_ref, o_ref):
    u = u_ref[...]
    msk = lax.broadcasted_iota(jnp.int32, u.shape, 0) < _N
    um = jnp.where(msk, u, 0.0)
    m = jnp.sum(um, axis=0, keepdims=True) * (1.0 / _N)
    cz = jnp.where(msk, u - m, 0.0)
    v = jnp.sum(cz * cz, axis=0, keepdims=True) * (1.0 / _N)
    h = g_ref[...] * (cz * lax.rsqrt(v + _EPS)) + be_ref[...]
    o_ref[...] = jnp.dot(h, w_ref[...], preferred_element_type=jnp.float32)


def _bnmm(u, g, be, w):
    return pl.pallas_call(
        _bnmm_body,
        out_shape=jax.ShapeDtypeStruct((u.shape[0], w.shape[1]), jnp.float32),
    )(u, g, be, w)


def _fin_body(u_ref, w_ref, g_ref, be_ref, W3_ref, b3_ref, o_ref):
    u = u_ref[...]
    msk = lax.broadcasted_iota(jnp.int32, u.shape, 0) < _N
    um = jnp.where(msk, u, 0.0)
    m = jnp.sum(um, axis=0, keepdims=True) * (1.0 / _N)
    cz = jnp.where(msk, u - m, 0.0)
    v = jnp.sum(cz * cz, axis=0, keepdims=True) * (1.0 / _N)
    h = g_ref[...] * ((w_ref[...] - m) * lax.rsqrt(v + _EPS)) + be_ref[...]
    o_ref[...] = jnp.dot(h, W3_ref[...],
                         preferred_element_type=jnp.float32) + b3_ref[...]


def _fin(u, w, g, be, W3, b3):
    return pl.pallas_call(
        _fin_body,
        out_shape=jax.ShapeDtypeStruct((u.shape[0], W3.shape[1]), jnp.float32),
    )(u, w, g, be, W3, b3)


@functools.cache
def _make_sc_avg(D, k, scale):
    mesh = plsc.VectorSubcoreMesh(core_axis_name="c", subcore_axis_name="s",
                                  num_cores=_NC, num_subcores=_NS)
    steps = k * _NCH
    scratch = (
        [pltpu.VMEM((_BW,), jnp.int32) for _ in range(k)]
        + [pltpu.VMEM((_BW, D), jnp.float32),
           pltpu.VMEM((_CH, D), jnp.float32),
           pltpu.VMEM((_CH, D), jnp.float32),
           pltpu.SemaphoreType.DMA,
           pltpu.SemaphoreType.DMA]
    )

    @functools.partial(
        pl.kernel, mesh=mesh,
        out_type=jax.ShapeDtypeStruct((_NPAD, D), jnp.float32),
        scratch_types=scratch)
    def f(y_hbm, idx_hbm, out_hbm, *rest):
        idx_vs = rest[:k]
        acc, nb0, nb1, sem0, sem1 = rest[k:]
        wid = lax.axis_index("s") * _NC + lax.axis_index("c")
        base = wid * _BW
        for j in range(k):
            pltpu.sync_copy(idx_hbm.at[pl.ds(j * _NPAD + base, _BW)],
                            idx_vs[j])
        nbufs = (nb0, nb1)
        sems = (sem0, sem1)

        def fire(s):
            j, q = s // _NCH, s % _NCH
            return pltpu.async_copy(
                y_hbm.at[idx_vs[j].at[pl.ds(q * _CH, _CH)]],
                nbufs[s % 2], sems[s % 2])

        cps = {0: fire(0)}
        pltpu.sync_copy(y_hbm.at[pl.ds(base, _BW)], acc)
        for s in range(steps):
            if s + 1 < steps:
                cps[s + 1] = fire(s + 1)
            cps.pop(s).wait()
            nb = nbufs[s % 2]
            q = s % _NCH
            r0 = q * _CH

            def add_row(r, _):
                for c in range(D // _L):
                    sl = pl.ds(c * _L, _L)
                    acc[r, sl] = acc[r, sl] + nb[r - r0, sl]
                return 0

            lax.fori_loop(r0, r0 + _CH, add_row, 0)

        def scale_row(r, _):
            for c in range(D // _L):
                sl = pl.ds(c * _L, _L)
                acc[r, sl] = acc[r, sl] * scale
            return 0

        lax.fori_loop(0, _BW, scale_row, 0)
        pltpu.sync_copy(acc, out_hbm.at[pl.ds(base, _BW)])

    return f


def _padc(a, cols):
    return jnp.concatenate(
        [a, jnp.zeros((a.shape[0], cols - a.shape[1]), a.dtype)], axis=1)


def _padr(a, rows):
    return jnp.concatenate(
        [a, jnp.zeros((rows - a.shape[0], a.shape[1]), a.dtype)], axis=0)


def kernel(x, pos, W0, b0, W1, b1, W2, b2, W3, b3, g0, be0, g1, be1, g2, be2):
    idx6 = _knn_idx(pos)
    idxF = jnp.concatenate(
        [idx6.T, jnp.zeros((8, _NPAD - _N), jnp.int32)], axis=1).reshape(-1)

    xp = _padr(x, _NPAD)
    W0p = _padc(W0, 128)
    g0p, be0p = _padc(g0[None, :], 128), _padc(be0[None, :], 128)
    W1p = _padc(_padr(W1, 128), 128)
    g1p, be1p = _padc(g1[None, :], 128), _padc(be1[None, :], 128)
    W2p = _padc(_padr(W2, 128), 128)
    g2p, be2p = _padc(g2[None, :], 128), _padc(be2[None, :], 128)
    W3p = _padc(_padr(W3, 128), 8)
    b3p = _padc(b3[None, :], 8)

    g_avg1 = _make_sc_avg(128, 1, 0.5)
    g_avg6 = _make_sc_avg(128, 6, 1.0 / 7.0)
    y0 = _mm(xp, W0p)
    u0 = g_avg1(y0, idxF)
    y1 = _bnmm(u0, g0p, be0p, W1p)
    u1 = g_avg1(y1, idxF)
    y2 = _bnmm(u1, g1p, be1p, W2p)
    u2 = g_avg6(y2, idxF)
    w3 = g_avg1(u2, idxF)
    o = _fin(u2, w3, g2p, be2p, W3p, b3p)
    return o[:_N, :1]


# speedup vs baseline: 6.5096x; 1.5696x over previous
import functools

import jax
import jax.numpy as jnp
from jax import lax
from jax.experimental import pallas as pl
from jax.experimental.pallas import tpu as pltpu
from jax.experimental.pallas import tpu_sc as plsc

_N = 10000
_NPAD = 10240
_RB = 400
_EPS = 1e-5

_NC, _NS, _L = 2, 16, 16
_NW = _NC * _NS
_BW = _NPAD // _NW
_CH = 64
_NCH = _BW // _CH


def _knn_body(pr2_ref, sqr_ref, pc2_ref, sqc_ref, idx_ref, cross_scr):
    cross_scr[...] = jnp.dot(pr2_ref[...], pc2_ref[...],
                             preferred_element_type=jnp.float32)
    lane = lax.broadcasted_iota(jnp.int32, (8, 128), 1)
    inf = jnp.float32(jnp.inf)

    def row_group(g, _):
        r0 = pl.multiple_of(g * 8, 8)
        sqr8 = sqr_ref[pl.ds(r0, 8), :]

        def chunk(t, carry):
            states = []
            for h, (mv, iv) in enumerate(carry):
                s = 2 * t + h
                c0 = pl.multiple_of(s * 128, 128)
                cr = cross_scr[pl.ds(r0, 8), pl.ds(c0, 128)]
                sc = sqc_ref[:, pl.ds(c0, 128)]
                nv = (sqr8 + sc) - 2.0 * cr
                ni = lane + s * 128
                mv2, iv2 = [], []
                for l in range(6):
                    c = nv < mv[l]
                    mv2.append(jnp.where(c, nv, mv[l]))
                    iv2.append(jnp.where(c, ni, iv[l]))
                    nv, ni = (jnp.where(c, mv[l], nv),
                              jnp.where(c, iv[l], ni))
                states.append((tuple(mv2), tuple(iv2)))
            return tuple(states)

        m0 = tuple(jnp.full((8, 128), inf, jnp.float32) for _ in range(6))
        i0 = tuple(jnp.full((8, 128), 2 ** 30, jnp.int32) for _ in range(6))
        (mva, iva), (mvb, ivb) = lax.fori_loop(
            0, _NPAD // 256, chunk, ((m0, i0), (m0, i0)))

        vals = jnp.concatenate(mva + mvb, axis=1)
        idxs = jnp.concatenate(iva + ivb, axis=1)
        for t in range(6):
            m = jnp.min(vals, axis=1, keepdims=True)
            am = jnp.min(jnp.where(vals == m, idxs, jnp.int32(2 ** 30)),
                         axis=1, keepdims=True)
            idx_ref[pl.ds(r0, 8), t:t + 1] = am
            if t < 5:
                vals = jnp.where((vals == m) & (idxs == am), inf, vals)
        idx_ref[pl.ds(r0, 8), 6:7] = jnp.zeros((8, 1), jnp.int32)
        idx_ref[pl.ds(r0, 8), 7:8] = jnp.zeros((8, 1), jnp.int32)
        return 0

    lax.fori_loop(0, _RB // 8, row_group, 0)


_knn_call = pl.pallas_call(
    _knn_body,
    grid=(_N // _RB,),
    in_specs=[
        pl.BlockSpec((_RB, 2), lambda i: (i, 0)),
        pl.BlockSpec((_RB, 1), lambda i: (i, 0)),
        pl.BlockSpec((2, _NPAD), lambda i: (0, 0)),
        pl.BlockSpec((1, _NPAD), lambda i: (0, 0)),
    ],
    out_specs=pl.BlockSpec((_RB, 8), lambda i: (i, 0)),
    out_shape=jax.ShapeDtypeStruct((_N, 8), jnp.int32),
    scratch_shapes=[pltpu.VMEM((_RB, _NPAD), jnp.float32)],
)


def _knn_idx(pos):
    p = pos[:, :2]
    sq = jnp.sum(p * p, axis=1)
    pc2 = jnp.concatenate(
        [p.T, jnp.zeros((2, _NPAD - _N), jnp.float32)], axis=1)
    sqc = jnp.concatenate(
        [sq[None, :], jnp.full((1, _NPAD - _N), jnp.inf, jnp.float32)], axis=1)
    return _knn_call(p, sq[:, None], pc2, sqc)


def _mm_body(x_ref, w_ref, o_ref):
    o_ref[...] = jnp.dot(x_ref[...], w_ref[...],
                         preferred_element_type=jnp.float32)


def _mm(x, w):
    return pl.pallas_call(
        _mm_body,
        out_shape=jax.ShapeDtypeStruct((x.shape[0], w.shape[1]), jnp.float32),
    )(x, w)


def _bnmm_body(u_ref, g_ref, be_ref, w_ref, o_ref):
    u = u_ref[...]
    msk = lax.broadcasted_iota(jnp.int32, u.shape, 0) < _N
    um = jnp.where(msk, u, 0.0)
    m = jnp.sum(um, axis=0, keepdims=True) * (1.0 / _N)
    cz = jnp.where(msk, u - m, 0.0)
    v = jnp.sum(cz * cz, axis=0, keepdims=True) * (1.0 / _N)
    h = g_ref[...] * (cz * lax.rsqrt(v + _EPS)) + be_ref[...]
    o_ref[...] = jnp.dot(h, w_ref[...], preferred_element_type=jnp.float32)


def _bnmm(u, g, be, w):
    return pl.pallas_call(
        _bnmm_body,
        out_shape=jax.ShapeDtypeStruct((u.shape[0], w.shape[1]), jnp.float32),
    )(u, g, be, w)


def _fin_body(u_ref, w_ref, g_ref, be_ref, W3_ref, b3_ref, o_ref):
    u = u_ref[...]
    msk = lax.broadcasted_iota(jnp.int32, u.shape, 0) < _N
    um = jnp.where(msk, u, 0.0)
    m = jnp.sum(um, axis=0, keepdims=True) * (1.0 / _N)
    cz = jnp.where(msk, u - m, 0.0)
    v = jnp.sum(cz * cz, axis=0, keepdims=True) * (1.0 / _N)
    h = g_ref[...] * ((w_ref[...] - m) * lax.rsqrt(v + _EPS)) + be_ref[...]
    o_ref[...] = jnp.dot(h, W3_ref[...],
                         preferred_element_type=jnp.float32) + b3_ref[...]


def _fin(u, w, g, be, W3, b3):
    return pl.pallas_call(
        _fin_body,
        out_shape=jax.ShapeDtypeStruct((u.shape[0], W3.shape[1]), jnp.float32),
    )(u, w, g, be, W3, b3)


@functools.cache
def _make_sc_avg(D, k, scale):
    mesh = plsc.VectorSubcoreMesh(core_axis_name="c", subcore_axis_name="s",
                                  num_cores=_NC, num_subcores=_NS)
    steps = k * _NCH
    scratch = (
        [pltpu.VMEM((_BW,), jnp.int32) for _ in range(k)]
        + [pltpu.VMEM((_BW, D), jnp.float32),
           pltpu.VMEM((_CH, D), jnp.float32),
           pltpu.VMEM((_CH, D), jnp.float32),
           pltpu.SemaphoreType.DMA,
           pltpu.SemaphoreType.DMA]
    )

    @functools.partial(
        pl.kernel, mesh=mesh,
        out_type=jax.ShapeDtypeStruct((_NPAD, D), jnp.float32),
        scratch_types=scratch)
    def f(y_hbm, idx_hbm, out_hbm, *rest):
        idx_vs = rest[:k]
        acc, nb0, nb1, sem0, sem1 = rest[k:]
        wid = lax.axis_index("s") * _NC + lax.axis_index("c")
        base = wid * _BW
        for j in range(k):
            pltpu.sync_copy(idx_hbm.at[pl.ds(j * _NPAD + base, _BW)],
                            idx_vs[j])
        nbufs = (nb0, nb1)
        sems = (sem0, sem1)

        def fire(s):
            j, q = s // _NCH, s % _NCH
            return pltpu.async_copy(
                y_hbm.at[idx_vs[j].at[pl.ds(q * _CH, _CH)]],
                nbufs[s % 2], sems[s % 2])

        cps = {0: fire(0)}
        pltpu.sync_copy(y_hbm.at[pl.ds(base, _BW)], acc)
        for s in range(steps):
            if s + 1 < steps:
                cps[s + 1] = fire(s + 1)
            cps.pop(s).wait()
            nb = nbufs[s % 2]
            q = s % _NCH
            r0 = q * _CH

            def add_row(r, _):
                for c in range(D // _L):
                    sl = pl.ds(c * _L, _L)
                    acc[r, sl] = acc[r, sl] + nb[r - r0, sl]
                return 0

            lax.fori_loop(r0, r0 + _CH, add_row, 0)

        def scale_row(r, _):
            for c in range(D // _L):
                sl = pl.ds(c * _L, _L)
                acc[r, sl] = acc[r, sl] * scale
            return 0

        lax.fori_loop(0, _BW, scale_row, 0)
        pltpu.sync_copy(acc, out_hbm.at[pl.ds(base, _BW)])

    return f


def _padc(a, cols):
    return jnp.concatenate(
        [a, jnp.zeros((a.shape[0], cols - a.shape[1]), a.dtype)], axis=1)


def _padr(a, rows):
    return jnp.concatenate(
        [a, jnp.zeros((rows - a.shape[0], a.shape[1]), a.dtype)], axis=0)


def kernel(x, pos, W0, b0, W1, b1, W2, b2, W3, b3, g0, be0, g1, be1, g2, be2):
    idx6 = _knn_idx(pos)
    idxF = jnp.concatenate(
        [idx6.T, jnp.zeros((8, _NPAD - _N), jnp.int32)], axis=1).reshape(-1)

    xp = _padr(x, _NPAD)
    W0p = _padc(W0, 128)
    g0p, be0p = _padc(g0[None, :], 128), _padc(be0[None, :], 128)
    W1p = _padc(_padr(W1, 128), 128)
    g1p, be1p = _padc(g1[None, :], 128), _padc(be1[None, :], 128)
    W2p = _padc(_padr(W2, 128), 128)
    g2p, be2p = _padc(g2[None, :], 128), _padc(be2[None, :], 128)
    W3p = _padc(_padr(W3, 128), 8)
    b3p = _padc(b3[None, :], 8)

    g_avg1 = _make_sc_avg(128, 1, 0.5)
    g_avg6 = _make_sc_avg(128, 6, 1.0 / 7.0)
    y0 = _mm(xp, W0p)
    u0 = g_avg1(y0, idxF)
    y1 = _bnmm(u0, g0p, be0p, W1p)
    u1 = g_avg1(y1, idxF)
    y2 = _bnmm(u1, g1p, be1p, W2p)
    u2 = g_avg6(y2, idxF)
    w3 = g_avg1(u2, idxF)
    o = _fin(u2, w3, g2p, be2p, W3p, b3p)
    return o[:_N, :1]
